# baseline (speedup 1.0000x reference)
import jax
import jax.numpy as jnp
from jax import lax
from jax.experimental import pallas as pl
from jax.experimental.pallas import tpu as pltpu
from jax.experimental.pallas import tpu_sc as plsc

B = 16384
K = 64
HK = 2 * K
NC = 2
NS = 16
NW = NC * NS
RPW = B // NW
CHUNK = 128
NCHUNK = RPW // CHUNK
GROUP = 16


def _mf_body(tx_hbm, u128_hbm, v128_hbm, bu_hbm, bi_hbm, bias_hbm,
             out_hbm,
             tx_v, uid_v, iid_v, uh_v, ih_v, ub, vb, bu_v, bi_v,
             out_v, bias_v, tile_v,
             sem_idx, sem_rows, sem_bias):
  c = lax.axis_index("c")
  s = lax.axis_index("s")
  wid = s * NC + c
  base = wid * RPW

  idx_copies = []
  for j in range(NCHUNK):
    idx_copies.append(pltpu.make_async_copy(
        tx_hbm.at[pl.ds(base + j * CHUNK, CHUNK), :], tx_v.at[j], sem_idx))
  idx_copies.append(pltpu.make_async_copy(bias_hbm, bias_v, sem_idx))
  for cp in idx_copies:
    cp.start()
  for cp in idx_copies:
    cp.wait()

  lane16 = lax.iota(jnp.int32, 16)
  zero16 = jnp.zeros((16,), jnp.int32)
  one16 = jnp.full((16,), 1, jnp.int32)
  for j in range(NCHUNK):
    j16 = jnp.full((16,), j, jnp.int32)
    for i in range(CHUNK // 16):
      rows = lane16 + i * 16
      sl = pl.ds(i * 16, 16)
      u16 = plsc.load_gather(tx_v, [j16, rows, zero16])
      i16 = plsc.load_gather(tx_v, [j16, rows, one16])
      uid_v[j, sl] = u16
      iid_v[j, sl] = i16
      uh_v[j, sl] = u16 >> 1
      ih_v[j, sl] = i16 >> 1

  def start_chunk(j):
    buf = j % 2
    cps = (pltpu.make_async_copy(u128_hbm.at[uh_v.at[j]], ub.at[buf],
                                 sem_rows),
           pltpu.make_async_copy(v128_hbm.at[ih_v.at[j]], vb.at[buf],
                                 sem_rows))
    for cp in cps:
      cp.start()
    return cps

  pending = start_chunk(0)

  bias_copies = []
  for j in range(NCHUNK):
    sl = pl.ds(j * CHUNK, CHUNK)
    bias_copies.append(pltpu.make_async_copy(
        bu_hbm.at[uid_v.at[j]], bu_v.at[sl], sem_bias))
    bias_copies.append(pltpu.make_async_copy(
        bi_hbm.at[iid_v.at[j]], bi_v.at[sl], sem_bias))
  for cp in bias_copies:
    cp.start()
  for cp in bias_copies:
    cp.wait()

  bias_vec = bias_v[...]
  lane = lax.iota(jnp.int32, 16)

  for j in range(NCHUNK):
    buf = j % 2
    for cp in pending:
      cp.wait()
    if j + 1 < NCHUNK:
      pending = start_chunk(j + 1)

    def group_body(g, carry, j=j, buf=buf):
      rbase = g * GROUP
      uid16 = uid_v[j, pl.ds(rbase, GROUP)]
      iid16 = iid_v[j, pl.ds(rbase, GROUP)]
      pu16 = uid16 & 1
      pi16 = iid16 & 1
      for rr in range(GROUP):
        row = rbase + rr
        mu = lax.broadcast(pu16[rr], (16,)) != 0
        mi = lax.broadcast(pi16[rr], (16,)) != 0
        acc = None
        for cb in range(K // 16):
          ulo = ub[buf, row, pl.ds(cb * 16, 16)]
          uhi = ub[buf, row, pl.ds(K + cb * 16, 16)]
          vlo = vb[buf, row, pl.ds(cb * 16, 16)]
          vhi = vb[buf, row, pl.ds(K + cb * 16, 16)]
          prod = jnp.where(mu, uhi, ulo) * jnp.where(mi, vhi, vlo)
          acc = prod if acc is None else acc + prod
        tile_v[pl.ds(rr * 16, 16)] = acc
      dotv = plsc.load_gather(tile_v, [lane * 16])
      for cb in range(1, 16):
        dotv = dotv + plsc.load_gather(tile_v, [lane * 16 + cb])
      osl = pl.ds(j * CHUNK + rbase, GROUP)
      out_v[osl] = dotv + bu_v[osl] + bi_v[osl] + bias_vec
      return carry

    lax.fori_loop(0, CHUNK // GROUP, group_body, 0)

  pltpu.sync_copy(out_v, out_hbm.at[pl.ds(base, RPW)])


@jax.jit
def kernel(train_x, user_w, item_w, bias_user_w, bias_item_w, bias):
  nu, k = user_w.shape
  ni, _ = item_w.shape
  u128 = user_w.reshape(nu // 2, 2 * k)
  v128 = item_w.reshape(ni // 2, 2 * k)
  bu = bias_user_w.reshape(-1)
  bi = bias_item_w.reshape(-1)
  bias16 = jnp.broadcast_to(bias, (16,))
  mesh = plsc.VectorSubcoreMesh(core_axis_name="c", subcore_axis_name="s",
                                num_cores=NC, num_subcores=NS)
  fn = pl.kernel(
      _mf_body,
      out_type=jax.ShapeDtypeStruct((B,), jnp.float32),
      mesh=mesh,
      compiler_params=pltpu.CompilerParams(needs_layout_passes=False,
                                           use_tc_tiling_on_sc=False),
      scratch_types=[
          pltpu.VMEM((NCHUNK, CHUNK, 2), jnp.int32),
          pltpu.VMEM((NCHUNK, CHUNK), jnp.int32),
          pltpu.VMEM((NCHUNK, CHUNK), jnp.int32),
          pltpu.VMEM((NCHUNK, CHUNK), jnp.int32),
          pltpu.VMEM((NCHUNK, CHUNK), jnp.int32),
          pltpu.VMEM((2, CHUNK, HK), jnp.float32),
          pltpu.VMEM((2, CHUNK, HK), jnp.float32),
          pltpu.VMEM((RPW,), jnp.float32),
          pltpu.VMEM((RPW,), jnp.float32),
          pltpu.VMEM((RPW,), jnp.float32),
          pltpu.VMEM((16,), jnp.float32),
          pltpu.VMEM((GROUP * 16,), jnp.float32),
          pltpu.SemaphoreType.DMA,
          pltpu.SemaphoreType.DMA,
          pltpu.SemaphoreType.DMA,
      ],
  )
  return fn(train_x, u128, v128, bu, bi, bias16)


# speedup vs baseline: 1.2403x; 1.2403x over previous
import jax
import jax.numpy as jnp
from jax import lax
from jax.experimental import pallas as pl
from jax.experimental.pallas import tpu as pltpu
from jax.experimental.pallas import tpu_sc as plsc

B = 16384
K = 64
HK = 2 * K
NC = 2
NS = 16
NW = NC * NS
RPW = B // NW
CHUNK = 128
NCHUNK = RPW // CHUNK
GROUP = 16

BM = 512
NBLK = 98
SPLIT = BM * NBLK


def _pack_body(u0, u1, v0, v1, ou, ov):
  ou[...] = jnp.concatenate([u0[...].T, u1[...].T], axis=1)
  ov[...] = jnp.concatenate([v0[...].T, v1[...].T], axis=1)


def _pack_tables(user_t, item_t):
  in_spec0 = pl.BlockSpec((K, BM), lambda i: (0, i))
  in_spec1 = pl.BlockSpec((K, BM), lambda i: (0, i + NBLK))
  out_spec = pl.BlockSpec((BM, HK), lambda i: (i, 0))
  return pl.pallas_call(
      _pack_body,
      grid=(NBLK,),
      in_specs=[in_spec0, in_spec1, in_spec0, in_spec1],
      out_specs=[out_spec, out_spec],
      out_shape=[jax.ShapeDtypeStruct((SPLIT, HK), jnp.float32),
                 jax.ShapeDtypeStruct((SPLIT, HK), jnp.float32)],
  )(user_t, user_t, item_t, item_t)


def _mf_body(uid_hbm, iid_hbm, u128_hbm, v128_hbm, bu_hbm, bi_hbm, bias_hbm,
             out_hbm,
             uid_v, iid_v, uh_v, ih_v, ub, vb, bu_v, bi_v,
             out_v, bias_v, tile_v,
             sem_idx, sem_rows, sem_bias):
  c = lax.axis_index("c")
  s = lax.axis_index("s")
  wid = s * NC + c
  base = wid * RPW

  idx_copies = []
  for j in range(NCHUNK):
    sl = pl.ds(base + j * CHUNK, CHUNK)
    idx_copies.append(pltpu.make_async_copy(uid_hbm.at[sl], uid_v.at[j],
                                            sem_idx))
    idx_copies.append(pltpu.make_async_copy(iid_hbm.at[sl], iid_v.at[j],
                                            sem_idx))
  idx_copies.append(pltpu.make_async_copy(bias_hbm, bias_v, sem_idx))
  for cp in idx_copies:
    cp.start()
  for cp in idx_copies:
    cp.wait()

  for j in range(NCHUNK):
    for i in range(CHUNK // 16):
      sl = pl.ds(i * 16, 16)
      u16 = uid_v[j, sl]
      i16 = iid_v[j, sl]
      uh_v[j, sl] = u16 - (u16 >= SPLIT).astype(jnp.int32) * SPLIT
      ih_v[j, sl] = i16 - (i16 >= SPLIT).astype(jnp.int32) * SPLIT

  def start_chunk(j):
    buf = j % 2
    cps = (pltpu.make_async_copy(u128_hbm.at[uh_v.at[j]], ub.at[buf],
                                 sem_rows),
           pltpu.make_async_copy(v128_hbm.at[ih_v.at[j]], vb.at[buf],
                                 sem_rows))
    for cp in cps:
      cp.start()
    return cps

  pending = start_chunk(0)

  bias_copies = []
  for j in range(NCHUNK):
    sl = pl.ds(j * CHUNK, CHUNK)
    bias_copies.append(pltpu.make_async_copy(
        bu_hbm.at[uid_v.at[j]], bu_v.at[sl], sem_bias))
    bias_copies.append(pltpu.make_async_copy(
        bi_hbm.at[iid_v.at[j]], bi_v.at[sl], sem_bias))
  for cp in bias_copies:
    cp.start()
  for cp in bias_copies:
    cp.wait()

  bias_vec = bias_v[...]
  lane = lax.iota(jnp.int32, 16)

  for j in range(NCHUNK):
    buf = j % 2
    for cp in pending:
      cp.wait()
    if j + 1 < NCHUNK:
      pending = start_chunk(j + 1)

    def group_body(g, carry, j=j, buf=buf):
      rbase = g * GROUP
      uid16 = uid_v[j, pl.ds(rbase, GROUP)]
      iid16 = iid_v[j, pl.ds(rbase, GROUP)]
      pu16 = (uid16 >= SPLIT).astype(jnp.int32)
      pi16 = (iid16 >= SPLIT).astype(jnp.int32)
      for rr in range(GROUP):
        row = rbase + rr
        mu = lax.broadcast(pu16[rr], (16,)) != 0
        mi = lax.broadcast(pi16[rr], (16,)) != 0
        acc = None
        for cb in range(K // 16):
          ulo = ub[buf, row, pl.ds(cb * 16, 16)]
          uhi = ub[buf, row, pl.ds(K + cb * 16, 16)]
          vlo = vb[buf, row, pl.ds(cb * 16, 16)]
          vhi = vb[buf, row, pl.ds(K + cb * 16, 16)]
          prod = jnp.where(mu, uhi, ulo) * jnp.where(mi, vhi, vlo)
          acc = prod if acc is None else acc + prod
        tile_v[pl.ds(rr * 16, 16)] = acc
      dotv = plsc.load_gather(tile_v, [lane * 16])
      for cb in range(1, 16):
        dotv = dotv + plsc.load_gather(tile_v, [lane * 16 + cb])
      osl = pl.ds(j * CHUNK + rbase, GROUP)
      out_v[osl] = dotv + bu_v[osl] + bi_v[osl] + bias_vec
      return carry

    lax.fori_loop(0, CHUNK // GROUP, group_body, 0)

  pltpu.sync_copy(out_v, out_hbm.at[pl.ds(base, RPW)])


@jax.jit
def kernel(train_x, user_w, item_w, bias_user_w, bias_item_w, bias):
  uid = train_x[:, 0]
  iid = train_x[:, 1]
  u128, v128 = _pack_tables(user_w.T, item_w.T)
  bu = bias_user_w.reshape(-1)
  bi = bias_item_w.reshape(-1)
  bias16 = jnp.broadcast_to(bias, (16,))
  mesh = plsc.VectorSubcoreMesh(core_axis_name="c", subcore_axis_name="s",
                                num_cores=NC, num_subcores=NS)
  fn = pl.kernel(
      _mf_body,
      out_type=jax.ShapeDtypeStruct((B,), jnp.float32),
      mesh=mesh,
      compiler_params=pltpu.CompilerParams(needs_layout_passes=False,
                                           use_tc_tiling_on_sc=False),
      scratch_types=[
          pltpu.VMEM((NCHUNK, CHUNK), jnp.int32),
          pltpu.VMEM((NCHUNK, CHUNK), jnp.int32),
          pltpu.VMEM((NCHUNK, CHUNK), jnp.int32),
          pltpu.VMEM((NCHUNK, CHUNK), jnp.int32),
          pltpu.VMEM((2, CHUNK, HK), jnp.float32),
          pltpu.VMEM((2, CHUNK, HK), jnp.float32),
          pltpu.VMEM((RPW,), jnp.float32),
          pltpu.VMEM((RPW,), jnp.float32),
          pltpu.VMEM((RPW,), jnp.float32),
          pltpu.VMEM((16,), jnp.float32),
          pltpu.VMEM((GROUP * 16,), jnp.float32),
          pltpu.SemaphoreType.DMA,
          pltpu.SemaphoreType.DMA,
          pltpu.SemaphoreType.DMA,
      ],
  )
  return fn(uid, iid, u128, v128, bu, bi, bias16)


# speedup vs baseline: 1.5225x; 1.2275x over previous
import jax
import jax.numpy as jnp
from jax import lax
from jax.experimental import pallas as pl
from jax.experimental.pallas import tpu as pltpu
from jax.experimental.pallas import tpu_sc as plsc

B = 16384
K = 64
HK = 2 * K
NC = 2
NS = 16
NW = NC * NS
RPW = B // NW
CHUNK = 128
NCHUNK = RPW // CHUNK
GROUP = 16

BM = 1024
NBLK = 49
SPLIT = BM * NBLK


def _pack_body(u0, u1, v0, v1, ou, ov):
  ident = (lax.broadcasted_iota(jnp.int32, (K, K), 0) ==
           lax.broadcasted_iota(jnp.int32, (K, K), 1)).astype(jnp.float32)
  dn = (((0,), (0,)), ((), ()))

  def t(x):
    return lax.dot_general(x[...], ident, dn,
                           preferred_element_type=jnp.float32)

  ou[...] = jnp.concatenate([t(u0), t(u1)], axis=1)
  ov[...] = jnp.concatenate([t(v0), t(v1)], axis=1)


def _pack_tables(user_t, item_t):
  in_spec0 = pl.BlockSpec((K, BM), lambda i: (0, i))
  in_spec1 = pl.BlockSpec((K, BM), lambda i: (0, i + NBLK))
  out_spec = pl.BlockSpec((BM, HK), lambda i: (i, 0))
  return pl.pallas_call(
      _pack_body,
      grid=(NBLK,),
      in_specs=[in_spec0, in_spec1, in_spec0, in_spec1],
      out_specs=[out_spec, out_spec],
      out_shape=[jax.ShapeDtypeStruct((SPLIT, HK), jnp.float32),
                 jax.ShapeDtypeStruct((SPLIT, HK), jnp.float32)],
  )(user_t, user_t, item_t, item_t)


def _mf_body(uid_hbm, iid_hbm, u128_hbm, v128_hbm, bu_hbm, bi_hbm, bias_hbm,
             out_hbm,
             uid_v, iid_v, uh_v, ih_v, ub, vb, bu_v, bi_v,
             out_v, bias_v, tile_v,
             sem_idx, sem_rows, sem_bias):
  c = lax.axis_index("c")
  s = lax.axis_index("s")
  wid = s * NC + c
  base = wid * RPW

  idx_copies = []
  for j in range(NCHUNK):
    sl = pl.ds(base + j * CHUNK, CHUNK)
    idx_copies.append(pltpu.make_async_copy(uid_hbm.at[sl], uid_v.at[j],
                                            sem_idx))
    idx_copies.append(pltpu.make_async_copy(iid_hbm.at[sl], iid_v.at[j],
                                            sem_idx))
  idx_copies.append(pltpu.make_async_copy(bias_hbm, bias_v, sem_idx))
  for cp in idx_copies:
    cp.start()
  for cp in idx_copies:
    cp.wait()

  for j in range(NCHUNK):
    for i in range(CHUNK // 16):
      sl = pl.ds(i * 16, 16)
      u16 = uid_v[j, sl]
      i16 = iid_v[j, sl]
      uh_v[j, sl] = u16 - (u16 >= SPLIT).astype(jnp.int32) * SPLIT
      ih_v[j, sl] = i16 - (i16 >= SPLIT).astype(jnp.int32) * SPLIT

  def start_chunk(j):
    buf = j % 2
    cps = (pltpu.make_async_copy(u128_hbm.at[uh_v.at[j]], ub.at[buf],
                                 sem_rows),
           pltpu.make_async_copy(v128_hbm.at[ih_v.at[j]], vb.at[buf],
                                 sem_rows))
    for cp in cps:
      cp.start()
    return cps

  pending = start_chunk(0)

  bias_copies = []
  for j in range(NCHUNK):
    sl = pl.ds(j * CHUNK, CHUNK)
    bias_copies.append(pltpu.make_async_copy(
        bu_hbm.at[uid_v.at[j]], bu_v.at[sl], sem_bias))
    bias_copies.append(pltpu.make_async_copy(
        bi_hbm.at[iid_v.at[j]], bi_v.at[sl], sem_bias))
  for cp in bias_copies:
    cp.start()
  for cp in bias_copies:
    cp.wait()

  bias_vec = bias_v[...]
  lane = lax.iota(jnp.int32, 16)

  for j in range(NCHUNK):
    buf = j % 2
    for cp in pending:
      cp.wait()
    if j + 1 < NCHUNK:
      pending = start_chunk(j + 1)

    def group_body(g, carry, j=j, buf=buf):
      rbase = g * GROUP
      uid16 = uid_v[j, pl.ds(rbase, GROUP)]
      iid16 = iid_v[j, pl.ds(rbase, GROUP)]
      pu16 = (uid16 >= SPLIT).astype(jnp.int32)
      pi16 = (iid16 >= SPLIT).astype(jnp.int32)
      for rr in range(GROUP):
        row = rbase + rr
        mu = lax.broadcast(pu16[rr], (16,)) != 0
        mi = lax.broadcast(pi16[rr], (16,)) != 0
        acc = None
        for cb in range(K // 16):
          ulo = ub[buf, row, pl.ds(cb * 16, 16)]
          uhi = ub[buf, row, pl.ds(K + cb * 16, 16)]
          vlo = vb[buf, row, pl.ds(cb * 16, 16)]
          vhi = vb[buf, row, pl.ds(K + cb * 16, 16)]
          prod = jnp.where(mu, uhi, ulo) * jnp.where(mi, vhi, vlo)
          acc = prod if acc is None else acc + prod
        tile_v[pl.ds(rr * 16, 16)] = acc
      dotv = plsc.load_gather(tile_v, [lane * 16])
      for cb in range(1, 16):
        dotv = dotv + plsc.load_gather(tile_v, [lane * 16 + cb])
      osl = pl.ds(j * CHUNK + rbase, GROUP)
      out_v[osl] = dotv + bu_v[osl] + bi_v[osl] + bias_vec
      return carry

    lax.fori_loop(0, CHUNK // GROUP, group_body, 0)

  pltpu.sync_copy(out_v, out_hbm.at[pl.ds(base, RPW)])


@jax.jit
def kernel(train_x, user_w, item_w, bias_user_w, bias_item_w, bias):
  uid = train_x[:, 0]
  iid = train_x[:, 1]
  u128, v128 = _pack_tables(user_w.T, item_w.T)
  bu = bias_user_w.reshape(-1)
  bi = bias_item_w.reshape(-1)
  bias16 = jnp.broadcast_to(bias, (16,))
  mesh = plsc.VectorSubcoreMesh(core_axis_name="c", subcore_axis_name="s",
                                num_cores=NC, num_subcores=NS)
  fn = pl.kernel(
      _mf_body,
      out_type=jax.ShapeDtypeStruct((B,), jnp.float32),
      mesh=mesh,
      compiler_params=pltpu.CompilerParams(needs_layout_passes=False,
                                           use_tc_tiling_on_sc=False),
      scratch_types=[
          pltpu.VMEM((NCHUNK, CHUNK), jnp.int32),
          pltpu.VMEM((NCHUNK, CHUNK), jnp.int32),
          pltpu.VMEM((NCHUNK, CHUNK), jnp.int32),
          pltpu.VMEM((NCHUNK, CHUNK), jnp.int32),
          pltpu.VMEM((2, CHUNK, HK), jnp.float32),
          pltpu.VMEM((2, CHUNK, HK), jnp.float32),
          pltpu.VMEM((RPW,), jnp.float32),
          pltpu.VMEM((RPW,), jnp.float32),
          pltpu.VMEM((RPW,), jnp.float32),
          pltpu.VMEM((16,), jnp.float32),
          pltpu.VMEM((GROUP * 16,), jnp.float32),
          pltpu.SemaphoreType.DMA,
          pltpu.SemaphoreType.DMA,
          pltpu.SemaphoreType.DMA,
      ],
  )
  return fn(uid, iid, u128, v128, bu, bi, bias16)
